# baseline (speedup 1.0000x reference)
import functools

import jax
import jax.numpy as jnp
from jax.experimental import pallas as pl
from jax.experimental.pallas import tpu as pltpu

_VMEM = dict(vmem_limit_bytes=100 * 1024 * 1024)


def _pad_hw(x):
    bn, H, W, C = x.shape
    zc = jnp.zeros((bn, H, 1, C), x.dtype)
    x = jnp.concatenate([zc, x, zc], axis=2)
    zr = jnp.zeros((bn, 1, W + 2, C), x.dtype)
    return jnp.concatenate([zr, x, zr], axis=1)


def _pool2(x):
    bn, H, W, C = x.shape
    x = x.reshape(bn, H // 2, 2, W, C)
    x = x[:, :, 0] + x[:, :, 1]
    x = x.reshape(bn, H // 2, W // 2, 2, C)
    return (x[:, :, :, 0] + x[:, :, :, 1]) * 0.25


def _tap_matmuls_val(x, w_ref, *, split=0):
    bn, Hp, Wp, Cin = x.shape
    H, W = Hp - 2, Wp - 2
    acc = None
    for t in range(9):
        dh, dw = divmod(t, 3)
        a = x[:, dh:dh + H, dw:dw + W, :].reshape(bn * H * W, Cin)
        if split:
            d = jnp.concatenate(
                [jnp.dot(a[:, :split], w_ref[t],
                         preferred_element_type=jnp.float32),
                 jnp.dot(a[:, split:], w_ref[t],
                         preferred_element_type=jnp.float32)], axis=1)
        else:
            d = jnp.dot(a, w_ref[t], preferred_element_type=jnp.float32)
        acc = d if acc is None else acc + d
    return acc


def _block_kernel(xp_ref, w1_ref, b1_ref, w2_ref, b2_ref, ws_ref, bs_ref,
                  o_ref, *, pre_relu, split2):
    bn, Hp, Wp, Cin = xp_ref.shape
    H, W = Hp - 2, Wp - 2
    c1 = w1_ref.shape[2]
    ct = o_ref.shape[-1]
    x = xp_ref[...]
    xr = jnp.maximum(x, 0) if pre_relu else x
    a1 = _tap_matmuls_val(xr, w1_ref) + b1_ref[...]
    h1 = jnp.maximum(a1, 0.0).astype(jnp.bfloat16)
    h1p = _pad_hw(h1.reshape(bn, H, W, c1))
    a2 = _tap_matmuls_val(h1p, w2_ref, split=split2) + b2_ref[...]
    h = _pool2(a2.reshape(bn, H, W, ct))
    px = _pool2(x[:, 1:H + 1, 1:W + 1, :].astype(jnp.float32))
    px = px.astype(jnp.bfloat16).reshape(bn * (H // 2) * (W // 2), Cin)
    sc = jnp.dot(px, ws_ref[...], preferred_element_type=jnp.float32) \
        + bs_ref[...]
    out = (h.reshape(-1, ct) + sc).astype(jnp.bfloat16)
    o_ref[...] = _pad_hw(out.reshape(bn, H // 2, W // 2, ct))


def _res_block(xp, w1, b1, w2, b2, ws, bs, cout, *, bi, pre_relu, split2=0):
    N, Hp, Wp, cin = xp.shape
    Ho, Wo = (Hp - 2) // 2 + 2, (Wp - 2) // 2 + 2
    return pl.pallas_call(
        functools.partial(_block_kernel, pre_relu=pre_relu, split2=split2),
        out_shape=jax.ShapeDtypeStruct((N, Ho, Wo, cout), jnp.bfloat16),
        grid=(N // bi,),
        in_specs=[
            pl.BlockSpec((bi, Hp, Wp, cin), lambda i: (i, 0, 0, 0)),
            pl.BlockSpec(w1.shape, lambda i: (0, 0, 0)),
            pl.BlockSpec((1, w1.shape[2]), lambda i: (0, 0)),
            pl.BlockSpec(w2.shape, lambda i: (0, 0, 0)),
            pl.BlockSpec((1, cout), lambda i: (0, 0)),
            pl.BlockSpec(ws.shape, lambda i: (0, 0)),
            pl.BlockSpec((1, cout), lambda i: (0, 0)),
        ],
        out_specs=pl.BlockSpec((bi, Ho, Wo, cout), lambda i: (i, 0, 0, 0)),
        compiler_params=pltpu.CompilerParams(
            dimension_semantics=("parallel",), **_VMEM),
    )(xp, w1, b1, w2, b2, ws, bs)


def _tap_matmuls(xp_ref, w_ref, *, pre_relu, split=0):
    bn, Hp, Wp, Cin = xp_ref.shape
    H, W = Hp - 2, Wp - 2
    acc = None
    for t in range(9):
        dh, dw = divmod(t, 3)
        a = xp_ref[:, dh:dh + H, dw:dw + W, :]
        if pre_relu:
            a = jnp.maximum(a, 0)
        a = a.reshape(bn * H * W, Cin)
        if split:
            d = jnp.concatenate(
                [jnp.dot(a[:, :split], w_ref[t],
                         preferred_element_type=jnp.float32),
                 jnp.dot(a[:, split:], w_ref[t],
                         preferred_element_type=jnp.float32)], axis=1)
        else:
            d = jnp.dot(a, w_ref[t], preferred_element_type=jnp.float32)
        acc = d if acc is None else acc + d
    return acc


def _conv1_kernel(xp_ref, w_ref, b_ref, *out_refs, pre_relu):
    o_ref = out_refs[0]
    px_ref = out_refs[1] if len(out_refs) > 1 else None
    bn, Hp, Wp, _ = xp_ref.shape
    H, W = Hp - 2, Wp - 2
    acc = _tap_matmuls(xp_ref, w_ref, pre_relu=pre_relu) + b_ref[...]
    out = jnp.maximum(acc, 0.0).astype(jnp.bfloat16)
    o_ref[...] = _pad_hw(out.reshape(bn, H, W, -1))
    if px_ref is not None:
        xin = xp_ref[:, 1:H + 1, 1:W + 1, :].astype(jnp.float32)
        px_ref[...] = _pool2(xin).astype(jnp.bfloat16)


def _conv2_pool_sc_kernel(hp_ref, px_ref, w_ref, b_ref, ws_ref, bs_ref,
                          o_ref, *, split, pool_sc):
    bn, Hp, Wp, _ = hp_ref.shape
    H, W = Hp - 2, Wp - 2
    ct = o_ref.shape[-1]
    acc = _tap_matmuls(hp_ref, w_ref, pre_relu=False, split=split) + b_ref[...]
    h = _pool2(acc.reshape(bn, H, W, ct))
    if pool_sc:
        xin = px_ref[:, 1:H + 1, 1:W + 1, :].astype(jnp.float32)
        px = _pool2(xin).astype(jnp.bfloat16)
    else:
        px = px_ref[...]
    cin = px.shape[-1]
    sc = jnp.dot(px.reshape(bn * (H // 2) * (W // 2), cin), ws_ref[...],
                 preferred_element_type=jnp.float32) + bs_ref[...]
    out = (h.reshape(-1, ct) + sc).astype(jnp.bfloat16)
    o_ref[...] = _pad_hw(out.reshape(bn, H // 2, W // 2, ct))


def _conv2_id_sum_kernel(hp_ref, xp_ref, w_ref, b_ref, o_ref):
    bn, Hp, Wp, _ = hp_ref.shape
    H, W = Hp - 2, Wp - 2
    ct = o_ref.shape[-1]
    acc = _tap_matmuls(hp_ref, w_ref, pre_relu=False) + b_ref[...]
    xin = xp_ref[:, 1:H + 1, 1:W + 1, :].astype(jnp.float32)
    s = jnp.maximum(acc + xin.reshape(bn * H * W, ct), 0.0)
    o_ref[...] = jnp.sum(s.reshape(bn, H * W, ct), axis=1)


def _head_kernel(x_ref, y_ref, wl_ref, bl_ref, wp_ref, bp_ref,
                 wa1_ref, ba1_ref, wa2_ref, ba2_ref, adv_ref, aux_ref):
    x = x_ref[...]
    adv = jnp.sum(x * wl_ref[...], axis=1, keepdims=True) + bl_ref[...]
    yp = jnp.dot(y_ref[...], wp_ref[...],
                 preferred_element_type=jnp.float32) + bp_ref[...]
    adv = adv + jnp.sum(x * yp, axis=1, keepdims=True)
    adv_ref[...] = adv
    h = jnp.maximum(
        jnp.dot(x, wa1_ref[...], preferred_element_type=jnp.float32)
        + ba1_ref[...], 0.0)
    aux_ref[...] = jnp.dot(h, wa2_ref[...],
                           preferred_element_type=jnp.float32) + ba2_ref[...]


def _conv1(xp, w9, b, cout, *, bi, ct, pre_relu, emit_px=False):
    N, Hp, Wp, cin = xp.shape
    cw = w9.shape[1]
    grid = (N // bi, cout // ct)
    out_shape = [jax.ShapeDtypeStruct((N, Hp, Wp, cout), jnp.bfloat16)]
    out_specs = [pl.BlockSpec((bi, Hp, Wp, ct), lambda i, j: (i, 0, 0, j))]
    if emit_px:
        Ho, Wo = (Hp - 2) // 2, (Wp - 2) // 2
        out_shape.append(jax.ShapeDtypeStruct((N, Ho, Wo, cin), jnp.bfloat16))
        out_specs.append(pl.BlockSpec((bi, Ho, Wo, cin),
                                      lambda i, j: (i, 0, 0, 0)))
    res = pl.pallas_call(
        functools.partial(_conv1_kernel, pre_relu=pre_relu),
        out_shape=tuple(out_shape),
        grid=grid,
        in_specs=[
            pl.BlockSpec((bi, Hp, Wp, cin), lambda i, j: (i, 0, 0, 0)),
            pl.BlockSpec((9, cw, ct), lambda i, j: (0, 0, j)),
            pl.BlockSpec((1, ct), lambda i, j: (0, j)),
        ],
        out_specs=tuple(out_specs),
        compiler_params=pltpu.CompilerParams(
            dimension_semantics=("parallel", "parallel"), **_VMEM),
    )(xp, w9, b)
    return res if emit_px else res[0]


def _conv2_pool_sc(hp, px, w9, b, ws, bs, cout, *, bi, ct, split=0,
                   pool_sc=False):
    N, Hp, Wp, c1 = hp.shape
    cin = px.shape[-1]
    cw = w9.shape[1]
    ctw = w9.shape[2] if split else ct
    Ho, Wo = (Hp - 2) // 2 + 2, (Wp - 2) // 2 + 2
    if pool_sc:
        px_spec = pl.BlockSpec((bi, Hp, Wp, cin), lambda i, j: (i, 0, 0, 0))
    else:
        px_spec = pl.BlockSpec((bi, Ho - 2, Wo - 2, cin),
                               lambda i, j: (i, 0, 0, 0))
    grid = (N // bi, cout // ct)
    return pl.pallas_call(
        functools.partial(_conv2_pool_sc_kernel, split=split, pool_sc=pool_sc),
        out_shape=jax.ShapeDtypeStruct((N, Ho, Wo, cout), jnp.bfloat16),
        grid=grid,
        in_specs=[
            pl.BlockSpec((bi, Hp, Wp, c1), lambda i, j: (i, 0, 0, 0)),
            px_spec,
            pl.BlockSpec((9, cw, ctw), lambda i, j: (0, 0, j)),
            pl.BlockSpec((1, ct), lambda i, j: (0, j)),
            pl.BlockSpec((cin, ct), lambda i, j: (0, j)),
            pl.BlockSpec((1, ct), lambda i, j: (0, j)),
        ],
        out_specs=pl.BlockSpec((bi, Ho, Wo, ct), lambda i, j: (i, 0, 0, j)),
        compiler_params=pltpu.CompilerParams(
            dimension_semantics=("parallel", "parallel"), **_VMEM),
    )(hp, px, w9, b, ws, bs)


def _conv2_id_sum(hp, xp, w9, b, cout, *, bi, ct):
    N, Hp, Wp, c1 = hp.shape
    grid = (N // bi, cout // ct)
    return pl.pallas_call(
        _conv2_id_sum_kernel,
        out_shape=jax.ShapeDtypeStruct((N, cout), jnp.float32),
        grid=grid,
        in_specs=[
            pl.BlockSpec((bi, Hp, Wp, c1), lambda i, j: (i, 0, 0, 0)),
            pl.BlockSpec((bi, Hp, Wp, ct), lambda i, j: (i, 0, 0, j)),
            pl.BlockSpec((9, c1, ct), lambda i, j: (0, 0, j)),
            pl.BlockSpec((1, ct), lambda i, j: (0, j)),
        ],
        out_specs=pl.BlockSpec((bi, ct), lambda i, j: (i, j)),
        compiler_params=pltpu.CompilerParams(
            dimension_semantics=("parallel", "parallel"), **_VMEM),
    )(hp, xp, w9, b)


def _w9(wm, cin):
    return wm.reshape(9, cin, wm.shape[-1])


def _row(b):
    return b.reshape(1, -1).astype(jnp.float32)


def _rowp(b):
    r = b.reshape(1, -1).astype(jnp.float32)
    return jnp.concatenate([r, r], axis=1)


def _bd3(w9):
    _, ci, co = w9.shape
    z = jnp.zeros((9, 2 * ci, 2 * co), w9.dtype)
    return z.at[:, :ci, :co].set(w9).at[:, ci:, co:].set(w9)


def _bd2(w):
    ci, co = w.shape
    z = jnp.zeros((2 * ci, 2 * co), w.dtype)
    return z.at[:ci, :co].set(w).at[ci:, co:].set(w)


def kernel(rd1_w1m, rd1_b1, rd1_w2m, rd1_b2, rd1_wsm, rd1_bs,
           rd2_w1m, rd2_b1, rd2_w2m, rd2_b2, rd2_wsm, rd2_bs,
           rd3_w1m, rd3_b1, rd3_w2m, rd3_b2, rd3_wsm, rd3_bs,
           rd4_w1m, rd4_b1, rd4_w2m, rd4_b2, rd4_wsm, rd4_bs,
           rd5_w1m, rd5_b1, rd5_w2m, rd5_b2, rd5_wsm, rd5_bs,
           rd6_w1m, rd6_b1, rd6_w2m, rd6_b2,
           linear_w, linear_b, proj_w, proj_b,
           aux1_w, aux1_b, aux2_w, aux2_b,
           x_src, x_tgt, y):
    B = x_src.shape[0]
    xs = jnp.transpose(x_src, (0, 2, 3, 1)).astype(jnp.bfloat16)
    xt = jnp.transpose(x_tgt, (0, 2, 3, 1)).astype(jnp.bfloat16)
    x = jnp.concatenate([xs, xt], axis=3)
    xp0 = jnp.pad(x, ((0, 0), (1, 1), (1, 1), (0, 0)))

    o1 = _res_block(xp0, _bd3(_w9(rd1_w1m, 3)), _rowp(rd1_b1),
                    _bd3(_w9(rd1_w2m, 64)), _rowp(rd1_b2),
                    _bd2(rd1_wsm), _rowp(rd1_bs), 128, bi=2, pre_relu=False)
    o2 = _res_block(o1, _bd3(_w9(rd2_w1m, 64)), _rowp(rd2_b1),
                    _bd3(_w9(rd2_w2m, 128)), _rowp(rd2_b2),
                    _bd2(rd2_wsm), _rowp(rd2_bs), 256, bi=8, pre_relu=True)
    o3 = _res_block(o2, _bd3(_w9(rd3_w1m, 128)), _rowp(rd3_b1),
                    _w9(rd3_w2m, 256), _rowp(rd3_b2),
                    _bd2(rd3_wsm), _rowp(rd3_bs), 512, bi=8, pre_relu=True,
                    split2=256)

    d = o3[..., 256:] - o3[..., :256]

    o4 = _res_block(d, _w9(rd4_w1m, 256), _row(rd4_b1),
                    _w9(rd4_w2m, 512), _row(rd4_b2),
                    rd4_wsm, _row(rd4_bs), 512, bi=16, pre_relu=True)

    h = _conv1(o4, _w9(rd5_w1m, 512), _row(rd5_b1), 1024,
               bi=16, ct=256, pre_relu=True)
    o5 = _conv2_pool_sc(h, o4, _w9(rd5_w2m, 1024), _row(rd5_b2),
                        rd5_wsm, _row(rd5_bs), 1024, bi=16, ct=256,
                        pool_sc=True)

    h = _conv1(o5, _w9(rd6_w1m, 1024), _row(rd6_b1), 1024,
               bi=32, ct=256, pre_relu=True)
    xpool = _conv2_id_sum(h, o5, _w9(rd6_w2m, 1024), _row(rd6_b2), 1024,
                          bi=32, ct=256)

    adv, aux = pl.pallas_call(
        _head_kernel,
        out_shape=(jax.ShapeDtypeStruct((B, 1), jnp.float32),
                   jax.ShapeDtypeStruct((B, aux2_w.shape[1]), jnp.float32)),
        compiler_params=pltpu.CompilerParams(**_VMEM),
    )(xpool, y, linear_w, linear_b, proj_w, proj_b,
      aux1_w, aux1_b, aux2_w, aux2_b)
    return adv, aux


# speedup vs baseline: 1.1046x; 1.1046x over previous
import functools

import jax
import jax.numpy as jnp
from jax.experimental import pallas as pl
from jax.experimental.pallas import tpu as pltpu

_VMEM = dict(vmem_limit_bytes=100 * 1024 * 1024)


def _pad_hw(x):
    bn, H, W, C = x.shape
    zc = jnp.zeros((bn, H, 1, C), x.dtype)
    x = jnp.concatenate([zc, x, zc], axis=2)
    zr = jnp.zeros((bn, 1, W + 2, C), x.dtype)
    return jnp.concatenate([zr, x, zr], axis=1)


def _pool2(x):
    bn, H, W, C = x.shape
    x = x.reshape(bn, H // 2, 2, W, C)
    x = x[:, :, 0] + x[:, :, 1]
    x = x.reshape(bn, H // 2, W // 2, 2, C)
    return (x[:, :, :, 0] + x[:, :, :, 1]) * 0.25


def _tap_matmuls_val(x, w_ref, *, split=0):
    bn, Hp, Wp, Cin = x.shape
    H, W = Hp - 2, Wp - 2
    acc = None
    for t in range(9):
        dh, dw = divmod(t, 3)
        a = x[:, dh:dh + H, dw:dw + W, :].reshape(bn * H * W, Cin)
        if split:
            d = jnp.concatenate(
                [jnp.dot(a[:, :split], w_ref[t],
                         preferred_element_type=jnp.float32),
                 jnp.dot(a[:, split:], w_ref[t],
                         preferred_element_type=jnp.float32)], axis=1)
        else:
            d = jnp.dot(a, w_ref[t], preferred_element_type=jnp.float32)
        acc = d if acc is None else acc + d
    return acc


def _block_kernel(xp_ref, w1_ref, b1_ref, w2_ref, b2_ref, ws_ref, bs_ref,
                  o_ref, *, pre_relu, split2):
    bn, Hp, Wp, Cin = xp_ref.shape
    H, W = Hp - 2, Wp - 2
    c1 = w1_ref.shape[2]
    ct = o_ref.shape[-1]
    x = xp_ref[...]
    xr = jnp.maximum(x, 0) if pre_relu else x
    a1 = _tap_matmuls_val(xr, w1_ref) + b1_ref[...]
    h1 = jnp.maximum(a1, 0.0).astype(jnp.bfloat16)
    h1p = _pad_hw(h1.reshape(bn, H, W, c1))
    a2 = _tap_matmuls_val(h1p, w2_ref, split=split2) + b2_ref[...]
    h = _pool2(a2.reshape(bn, H, W, ct))
    px = _pool2(x[:, 1:H + 1, 1:W + 1, :].astype(jnp.float32))
    px = px.astype(jnp.bfloat16).reshape(bn * (H // 2) * (W // 2), Cin)
    sc = jnp.dot(px, ws_ref[...], preferred_element_type=jnp.float32) \
        + bs_ref[...]
    out = (h.reshape(-1, ct) + sc).astype(jnp.bfloat16)
    o_ref[...] = _pad_hw(out.reshape(bn, H // 2, W // 2, ct))


def _res_block(xp, w1, b1, w2, b2, ws, bs, cout, *, bi, pre_relu, split2=0):
    N, Hp, Wp, cin = xp.shape
    Ho, Wo = (Hp - 2) // 2 + 2, (Wp - 2) // 2 + 2
    return pl.pallas_call(
        functools.partial(_block_kernel, pre_relu=pre_relu, split2=split2),
        out_shape=jax.ShapeDtypeStruct((N, Ho, Wo, cout), jnp.bfloat16),
        grid=(N // bi,),
        in_specs=[
            pl.BlockSpec((bi, Hp, Wp, cin), lambda i: (i, 0, 0, 0)),
            pl.BlockSpec(w1.shape, lambda i: (0, 0, 0)),
            pl.BlockSpec((1, w1.shape[2]), lambda i: (0, 0)),
            pl.BlockSpec(w2.shape, lambda i: (0, 0, 0)),
            pl.BlockSpec((1, cout), lambda i: (0, 0)),
            pl.BlockSpec(ws.shape, lambda i: (0, 0)),
            pl.BlockSpec((1, cout), lambda i: (0, 0)),
        ],
        out_specs=pl.BlockSpec((bi, Ho, Wo, cout), lambda i: (i, 0, 0, 0)),
        compiler_params=pltpu.CompilerParams(
            dimension_semantics=("parallel",), **_VMEM),
    )(xp, w1, b1, w2, b2, ws, bs)


def _tap_matmuls(xp_ref, w_ref, *, pre_relu, split=0):
    bn, Hp, Wp, Cin = xp_ref.shape
    H, W = Hp - 2, Wp - 2
    acc = None
    for t in range(9):
        dh, dw = divmod(t, 3)
        a = xp_ref[:, dh:dh + H, dw:dw + W, :]
        if pre_relu:
            a = jnp.maximum(a, 0)
        a = a.reshape(bn * H * W, Cin)
        if split:
            d = jnp.concatenate(
                [jnp.dot(a[:, :split], w_ref[t],
                         preferred_element_type=jnp.float32),
                 jnp.dot(a[:, split:], w_ref[t],
                         preferred_element_type=jnp.float32)], axis=1)
        else:
            d = jnp.dot(a, w_ref[t], preferred_element_type=jnp.float32)
        acc = d if acc is None else acc + d
    return acc


def _conv1_kernel(xp_ref, w_ref, b_ref, *out_refs, pre_relu):
    o_ref = out_refs[0]
    px_ref = out_refs[1] if len(out_refs) > 1 else None
    bn, Hp, Wp, _ = xp_ref.shape
    H, W = Hp - 2, Wp - 2
    acc = _tap_matmuls(xp_ref, w_ref, pre_relu=pre_relu) + b_ref[...]
    out = jnp.maximum(acc, 0.0).astype(jnp.bfloat16)
    o_ref[...] = _pad_hw(out.reshape(bn, H, W, -1))
    if px_ref is not None:
        xin = xp_ref[:, 1:H + 1, 1:W + 1, :].astype(jnp.float32)
        px_ref[...] = _pool2(xin).astype(jnp.bfloat16)


def _conv2_pool_sc_kernel(hp_ref, px_ref, w_ref, b_ref, ws_ref, bs_ref,
                          o_ref, *, split, pool_sc):
    bn, Hp, Wp, _ = hp_ref.shape
    H, W = Hp - 2, Wp - 2
    ct = o_ref.shape[-1]
    acc = _tap_matmuls(hp_ref, w_ref, pre_relu=False, split=split) + b_ref[...]
    h = _pool2(acc.reshape(bn, H, W, ct))
    if pool_sc:
        xin = px_ref[:, 1:H + 1, 1:W + 1, :].astype(jnp.float32)
        px = _pool2(xin).astype(jnp.bfloat16)
    else:
        px = px_ref[...]
    cin = px.shape[-1]
    sc = jnp.dot(px.reshape(bn * (H // 2) * (W // 2), cin), ws_ref[...],
                 preferred_element_type=jnp.float32) + bs_ref[...]
    out = (h.reshape(-1, ct) + sc).astype(jnp.bfloat16)
    o_ref[...] = _pad_hw(out.reshape(bn, H // 2, W // 2, ct))


def _conv2_id_sum_kernel(hp_ref, xp_ref, w_ref, b_ref, o_ref):
    bn, Hp, Wp, _ = hp_ref.shape
    H, W = Hp - 2, Wp - 2
    ct = o_ref.shape[-1]
    acc = _tap_matmuls(hp_ref, w_ref, pre_relu=False) + b_ref[...]
    xin = xp_ref[:, 1:H + 1, 1:W + 1, :].astype(jnp.float32)
    s = jnp.maximum(acc + xin.reshape(bn * H * W, ct), 0.0)
    o_ref[...] = jnp.sum(s.reshape(bn, H * W, ct), axis=1)


def _head_kernel(x_ref, y_ref, wl_ref, bl_ref, wp_ref, bp_ref,
                 wa1_ref, ba1_ref, wa2_ref, ba2_ref, adv_ref, aux_ref):
    x = x_ref[...]
    adv = jnp.sum(x * wl_ref[...], axis=1, keepdims=True) + bl_ref[...]
    yp = jnp.dot(y_ref[...], wp_ref[...],
                 preferred_element_type=jnp.float32) + bp_ref[...]
    adv = adv + jnp.sum(x * yp, axis=1, keepdims=True)
    adv_ref[...] = adv
    h = jnp.maximum(
        jnp.dot(x, wa1_ref[...], preferred_element_type=jnp.float32)
        + ba1_ref[...], 0.0)
    aux_ref[...] = jnp.dot(h, wa2_ref[...],
                           preferred_element_type=jnp.float32) + ba2_ref[...]


def _conv1(xp, w9, b, cout, *, bi, ct, pre_relu, emit_px=False):
    N, Hp, Wp, cin = xp.shape
    cw = w9.shape[1]
    grid = (N // bi, cout // ct)
    out_shape = [jax.ShapeDtypeStruct((N, Hp, Wp, cout), jnp.bfloat16)]
    out_specs = [pl.BlockSpec((bi, Hp, Wp, ct), lambda i, j: (i, 0, 0, j))]
    if emit_px:
        Ho, Wo = (Hp - 2) // 2, (Wp - 2) // 2
        out_shape.append(jax.ShapeDtypeStruct((N, Ho, Wo, cin), jnp.bfloat16))
        out_specs.append(pl.BlockSpec((bi, Ho, Wo, cin),
                                      lambda i, j: (i, 0, 0, 0)))
    res = pl.pallas_call(
        functools.partial(_conv1_kernel, pre_relu=pre_relu),
        out_shape=tuple(out_shape),
        grid=grid,
        in_specs=[
            pl.BlockSpec((bi, Hp, Wp, cin), lambda i, j: (i, 0, 0, 0)),
            pl.BlockSpec((9, cw, ct), lambda i, j: (0, 0, j)),
            pl.BlockSpec((1, ct), lambda i, j: (0, j)),
        ],
        out_specs=tuple(out_specs),
        compiler_params=pltpu.CompilerParams(
            dimension_semantics=("parallel", "parallel"), **_VMEM),
    )(xp, w9, b)
    return res if emit_px else res[0]


def _conv2_pool_sc(hp, px, w9, b, ws, bs, cout, *, bi, ct, split=0,
                   pool_sc=False):
    N, Hp, Wp, c1 = hp.shape
    cin = px.shape[-1]
    cw = w9.shape[1]
    ctw = w9.shape[2] if split else ct
    Ho, Wo = (Hp - 2) // 2 + 2, (Wp - 2) // 2 + 2
    if pool_sc:
        px_spec = pl.BlockSpec((bi, Hp, Wp, cin), lambda i, j: (i, 0, 0, 0))
    else:
        px_spec = pl.BlockSpec((bi, Ho - 2, Wo - 2, cin),
                               lambda i, j: (i, 0, 0, 0))
    grid = (N // bi, cout // ct)
    return pl.pallas_call(
        functools.partial(_conv2_pool_sc_kernel, split=split, pool_sc=pool_sc),
        out_shape=jax.ShapeDtypeStruct((N, Ho, Wo, cout), jnp.bfloat16),
        grid=grid,
        in_specs=[
            pl.BlockSpec((bi, Hp, Wp, c1), lambda i, j: (i, 0, 0, 0)),
            px_spec,
            pl.BlockSpec((9, cw, ctw), lambda i, j: (0, 0, j)),
            pl.BlockSpec((1, ct), lambda i, j: (0, j)),
            pl.BlockSpec((cin, ct), lambda i, j: (0, j)),
            pl.BlockSpec((1, ct), lambda i, j: (0, j)),
        ],
        out_specs=pl.BlockSpec((bi, Ho, Wo, ct), lambda i, j: (i, 0, 0, j)),
        compiler_params=pltpu.CompilerParams(
            dimension_semantics=("parallel", "parallel"), **_VMEM),
    )(hp, px, w9, b, ws, bs)


def _conv2_id_sum(hp, xp, w9, b, cout, *, bi, ct):
    N, Hp, Wp, c1 = hp.shape
    grid = (N // bi, cout // ct)
    return pl.pallas_call(
        _conv2_id_sum_kernel,
        out_shape=jax.ShapeDtypeStruct((N, cout), jnp.float32),
        grid=grid,
        in_specs=[
            pl.BlockSpec((bi, Hp, Wp, c1), lambda i, j: (i, 0, 0, 0)),
            pl.BlockSpec((bi, Hp, Wp, ct), lambda i, j: (i, 0, 0, j)),
            pl.BlockSpec((9, c1, ct), lambda i, j: (0, 0, j)),
            pl.BlockSpec((1, ct), lambda i, j: (0, j)),
        ],
        out_specs=pl.BlockSpec((bi, ct), lambda i, j: (i, j)),
        compiler_params=pltpu.CompilerParams(
            dimension_semantics=("parallel", "parallel"), **_VMEM),
    )(hp, xp, w9, b)


def _w9(wm, cin):
    return wm.reshape(9, cin, wm.shape[-1])


def _row(b):
    return b.reshape(1, -1).astype(jnp.float32)


def _rowp(b):
    r = b.reshape(1, -1).astype(jnp.float32)
    return jnp.concatenate([r, r], axis=1)


def _bd3(w9):
    _, ci, co = w9.shape
    z = jnp.zeros((9, 2 * ci, 2 * co), w9.dtype)
    return z.at[:, :ci, :co].set(w9).at[:, ci:, co:].set(w9)


def _bd2(w):
    ci, co = w.shape
    z = jnp.zeros((2 * ci, 2 * co), w.dtype)
    return z.at[:ci, :co].set(w).at[ci:, co:].set(w)


def kernel(rd1_w1m, rd1_b1, rd1_w2m, rd1_b2, rd1_wsm, rd1_bs,
           rd2_w1m, rd2_b1, rd2_w2m, rd2_b2, rd2_wsm, rd2_bs,
           rd3_w1m, rd3_b1, rd3_w2m, rd3_b2, rd3_wsm, rd3_bs,
           rd4_w1m, rd4_b1, rd4_w2m, rd4_b2, rd4_wsm, rd4_bs,
           rd5_w1m, rd5_b1, rd5_w2m, rd5_b2, rd5_wsm, rd5_bs,
           rd6_w1m, rd6_b1, rd6_w2m, rd6_b2,
           linear_w, linear_b, proj_w, proj_b,
           aux1_w, aux1_b, aux2_w, aux2_b,
           x_src, x_tgt, y):
    B = x_src.shape[0]
    xs = jnp.transpose(x_src, (0, 2, 3, 1)).astype(jnp.bfloat16)
    xt = jnp.transpose(x_tgt, (0, 2, 3, 1)).astype(jnp.bfloat16)
    x = jnp.concatenate([xs, xt], axis=3)
    xp0 = jnp.pad(x, ((0, 0), (1, 1), (1, 1), (0, 0)))

    o1 = _res_block(xp0, _bd3(_w9(rd1_w1m, 3)), _rowp(rd1_b1),
                    _bd3(_w9(rd1_w2m, 64)), _rowp(rd1_b2),
                    _bd2(rd1_wsm), _rowp(rd1_bs), 128, bi=2, pre_relu=False)

    h, px = _conv1(o1, _bd3(_w9(rd2_w1m, 64)), _rowp(rd2_b1), 256,
                   bi=8, ct=256, pre_relu=True, emit_px=True)
    o2 = _conv2_pool_sc(h, px, _bd3(_w9(rd2_w2m, 128)),
                        _rowp(rd2_b2), _bd2(rd2_wsm), _rowp(rd2_bs),
                        256, bi=8, ct=256)

    h, px = _conv1(o2, _bd3(_w9(rd3_w1m, 128)), _rowp(rd3_b1), 512,
                   bi=8, ct=512, pre_relu=True, emit_px=True)
    o3 = _conv2_pool_sc(h, px, _w9(rd3_w2m, 256),
                        _rowp(rd3_b2), _bd2(rd3_wsm), _rowp(rd3_bs),
                        512, bi=8, ct=512, split=256)

    d = o3[..., 256:] - o3[..., :256]

    h = _conv1(d, _w9(rd4_w1m, 256), _row(rd4_b1), 512,
               bi=16, ct=256, pre_relu=True)
    o4 = _conv2_pool_sc(h, d, _w9(rd4_w2m, 512), _row(rd4_b2),
                        rd4_wsm, _row(rd4_bs), 512, bi=16, ct=256,
                        pool_sc=True)

    h = _conv1(o4, _w9(rd5_w1m, 512), _row(rd5_b1), 1024,
               bi=16, ct=256, pre_relu=True)
    o5 = _conv2_pool_sc(h, o4, _w9(rd5_w2m, 1024), _row(rd5_b2),
                        rd5_wsm, _row(rd5_bs), 1024, bi=16, ct=256,
                        pool_sc=True)

    h = _conv1(o5, _w9(rd6_w1m, 1024), _row(rd6_b1), 1024,
               bi=32, ct=256, pre_relu=True)
    xpool = _conv2_id_sum(h, o5, _w9(rd6_w2m, 1024), _row(rd6_b2), 1024,
                          bi=32, ct=256)

    adv, aux = pl.pallas_call(
        _head_kernel,
        out_shape=(jax.ShapeDtypeStruct((B, 1), jnp.float32),
                   jax.ShapeDtypeStruct((B, aux2_w.shape[1]), jnp.float32)),
        compiler_params=pltpu.CompilerParams(**_VMEM),
    )(xpool, y, linear_w, linear_b, proj_w, proj_b,
      aux1_w, aux1_b, aux2_w, aux2_b)
    return adv, aux


# speedup vs baseline: 1.1235x; 1.0171x over previous
import functools

import jax
import jax.numpy as jnp
from jax.experimental import pallas as pl
from jax.experimental.pallas import tpu as pltpu

_VMEM = dict(vmem_limit_bytes=100 * 1024 * 1024)


def _pad_hw(x):
    bn, H, W, C = x.shape
    zc = jnp.zeros((bn, H, 1, C), x.dtype)
    x = jnp.concatenate([zc, x, zc], axis=2)
    zr = jnp.zeros((bn, 1, W + 2, C), x.dtype)
    return jnp.concatenate([zr, x, zr], axis=1)


def _pool2(x):
    bn, H, W, C = x.shape
    x = x.reshape(bn, H // 2, 2, W, C)
    x = x[:, :, 0] + x[:, :, 1]
    x = x.reshape(bn, H // 2, W // 2, 2, C)
    return (x[:, :, :, 0] + x[:, :, :, 1]) * 0.25


def _tap_matmuls_val(x, w_ref, *, split=0):
    bn, Hp, Wp, Cin = x.shape
    H, W = Hp - 2, Wp - 2
    acc = None
    for t in range(9):
        dh, dw = divmod(t, 3)
        a = x[:, dh:dh + H, dw:dw + W, :].reshape(bn * H * W, Cin)
        if split:
            d = jnp.concatenate(
                [jnp.dot(a[:, :split], w_ref[t],
                         preferred_element_type=jnp.float32),
                 jnp.dot(a[:, split:], w_ref[t],
                         preferred_element_type=jnp.float32)], axis=1)
        else:
            d = jnp.dot(a, w_ref[t], preferred_element_type=jnp.float32)
        acc = d if acc is None else acc + d
    return acc


def _block_kernel(xp_ref, w1_ref, b1_ref, w2_ref, b2_ref, ws_ref, bs_ref,
                  o_ref, *, pre_relu, split2):
    bn, Hp, Wp, Cin = xp_ref.shape
    H, W = Hp - 2, Wp - 2
    c1 = w1_ref.shape[2]
    ct = o_ref.shape[-1]
    x = xp_ref[...]
    xr = jnp.maximum(x, 0) if pre_relu else x
    a1 = _tap_matmuls_val(xr, w1_ref) + b1_ref[...]
    h1 = jnp.maximum(a1, 0.0).astype(jnp.bfloat16)
    h1p = _pad_hw(h1.reshape(bn, H, W, c1))
    a2 = _tap_matmuls_val(h1p, w2_ref, split=split2) + b2_ref[...]
    h = _pool2(a2.reshape(bn, H, W, ct))
    px = _pool2(x[:, 1:H + 1, 1:W + 1, :].astype(jnp.float32))
    px = px.astype(jnp.bfloat16).reshape(bn * (H // 2) * (W // 2), Cin)
    sc = jnp.dot(px, ws_ref[...], preferred_element_type=jnp.float32) \
        + bs_ref[...]
    out = (h.reshape(-1, ct) + sc).astype(jnp.bfloat16)
    o_ref[...] = _pad_hw(out.reshape(bn, H // 2, W // 2, ct))


def _res_block(xp, w1, b1, w2, b2, ws, bs, cout, *, bi, pre_relu, split2=0):
    N, Hp, Wp, cin = xp.shape
    Ho, Wo = (Hp - 2) // 2 + 2, (Wp - 2) // 2 + 2
    return pl.pallas_call(
        functools.partial(_block_kernel, pre_relu=pre_relu, split2=split2),
        out_shape=jax.ShapeDtypeStruct((N, Ho, Wo, cout), jnp.bfloat16),
        grid=(N // bi,),
        in_specs=[
            pl.BlockSpec((bi, Hp, Wp, cin), lambda i: (i, 0, 0, 0)),
            pl.BlockSpec(w1.shape, lambda i: (0, 0, 0)),
            pl.BlockSpec((1, w1.shape[2]), lambda i: (0, 0)),
            pl.BlockSpec(w2.shape, lambda i: (0, 0, 0)),
            pl.BlockSpec((1, cout), lambda i: (0, 0)),
            pl.BlockSpec(ws.shape, lambda i: (0, 0)),
            pl.BlockSpec((1, cout), lambda i: (0, 0)),
        ],
        out_specs=pl.BlockSpec((bi, Ho, Wo, cout), lambda i: (i, 0, 0, 0)),
        compiler_params=pltpu.CompilerParams(
            dimension_semantics=("parallel",), **_VMEM),
    )(xp, w1, b1, w2, b2, ws, bs)


def _tap_matmuls(xp_ref, w_ref, *, pre_relu, split=0):
    bn, Hp, Wp, Cin = xp_ref.shape
    H, W = Hp - 2, Wp - 2
    acc = None
    for t in range(9):
        dh, dw = divmod(t, 3)
        a = xp_ref[:, dh:dh + H, dw:dw + W, :]
        if pre_relu:
            a = jnp.maximum(a, 0)
        a = a.reshape(bn * H * W, Cin)
        if split:
            d = jnp.concatenate(
                [jnp.dot(a[:, :split], w_ref[t],
                         preferred_element_type=jnp.float32),
                 jnp.dot(a[:, split:], w_ref[t],
                         preferred_element_type=jnp.float32)], axis=1)
        else:
            d = jnp.dot(a, w_ref[t], preferred_element_type=jnp.float32)
        acc = d if acc is None else acc + d
    return acc


def _conv1_kernel(xp_ref, w_ref, b_ref, *out_refs, pre_relu):
    o_ref = out_refs[0]
    px_ref = out_refs[1] if len(out_refs) > 1 else None
    bn, Hp, Wp, _ = xp_ref.shape
    H, W = Hp - 2, Wp - 2
    acc = _tap_matmuls(xp_ref, w_ref, pre_relu=pre_relu) + b_ref[...]
    out = jnp.maximum(acc, 0.0).astype(jnp.bfloat16)
    o_ref[...] = _pad_hw(out.reshape(bn, H, W, -1))
    if px_ref is not None:
        xin = xp_ref[:, 1:H + 1, 1:W + 1, :].astype(jnp.float32)
        px_ref[...] = _pool2(xin).astype(jnp.bfloat16)


def _conv2_pool_sc_kernel(hp_ref, px_ref, w_ref, b_ref, ws_ref, bs_ref,
                          o_ref, *, split, pool_sc):
    bn, Hp, Wp, _ = hp_ref.shape
    H, W = Hp - 2, Wp - 2
    ct = o_ref.shape[-1]
    acc = _tap_matmuls(hp_ref, w_ref, pre_relu=False, split=split) + b_ref[...]
    h = _pool2(acc.reshape(bn, H, W, ct))
    if pool_sc:
        xin = px_ref[:, 1:H + 1, 1:W + 1, :].astype(jnp.float32)
        px = _pool2(xin).astype(jnp.bfloat16)
    else:
        px = px_ref[...]
    cin = px.shape[-1]
    sc = jnp.dot(px.reshape(bn * (H // 2) * (W // 2), cin), ws_ref[...],
                 preferred_element_type=jnp.float32) + bs_ref[...]
    out = (h.reshape(-1, ct) + sc).astype(jnp.bfloat16)
    o_ref[...] = _pad_hw(out.reshape(bn, H // 2, W // 2, ct))


def _conv2_id_sum_kernel(hp_ref, xp_ref, w_ref, b_ref, o_ref):
    bn, Hp, Wp, _ = hp_ref.shape
    H, W = Hp - 2, Wp - 2
    ct = o_ref.shape[-1]
    acc = _tap_matmuls(hp_ref, w_ref, pre_relu=False) + b_ref[...]
    xin = xp_ref[:, 1:H + 1, 1:W + 1, :].astype(jnp.float32)
    s = jnp.maximum(acc + xin.reshape(bn * H * W, ct), 0.0)
    o_ref[...] = jnp.sum(s.reshape(bn, H * W, ct), axis=1)


def _head_kernel(x_ref, y_ref, wl_ref, bl_ref, wp_ref, bp_ref,
                 wa1_ref, ba1_ref, wa2_ref, ba2_ref, adv_ref, aux_ref):
    x = x_ref[...]
    adv = jnp.sum(x * wl_ref[...], axis=1, keepdims=True) + bl_ref[...]
    yp = jnp.dot(y_ref[...], wp_ref[...],
                 preferred_element_type=jnp.float32) + bp_ref[...]
    adv = adv + jnp.sum(x * yp, axis=1, keepdims=True)
    adv_ref[...] = adv
    h = jnp.maximum(
        jnp.dot(x, wa1_ref[...], preferred_element_type=jnp.float32)
        + ba1_ref[...], 0.0)
    aux_ref[...] = jnp.dot(h, wa2_ref[...],
                           preferred_element_type=jnp.float32) + ba2_ref[...]


def _conv1(xp, w9, b, cout, *, bi, ct, pre_relu, emit_px=False):
    N, Hp, Wp, cin = xp.shape
    cw = w9.shape[1]
    grid = (N // bi, cout // ct)
    out_shape = [jax.ShapeDtypeStruct((N, Hp, Wp, cout), jnp.bfloat16)]
    out_specs = [pl.BlockSpec((bi, Hp, Wp, ct), lambda i, j: (i, 0, 0, j))]
    if emit_px:
        Ho, Wo = (Hp - 2) // 2, (Wp - 2) // 2
        out_shape.append(jax.ShapeDtypeStruct((N, Ho, Wo, cin), jnp.bfloat16))
        out_specs.append(pl.BlockSpec((bi, Ho, Wo, cin),
                                      lambda i, j: (i, 0, 0, 0)))
    res = pl.pallas_call(
        functools.partial(_conv1_kernel, pre_relu=pre_relu),
        out_shape=tuple(out_shape),
        grid=grid,
        in_specs=[
            pl.BlockSpec((bi, Hp, Wp, cin), lambda i, j: (i, 0, 0, 0)),
            pl.BlockSpec((9, cw, ct), lambda i, j: (0, 0, j)),
            pl.BlockSpec((1, ct), lambda i, j: (0, j)),
        ],
        out_specs=tuple(out_specs),
        compiler_params=pltpu.CompilerParams(
            dimension_semantics=("parallel", "parallel"), **_VMEM),
    )(xp, w9, b)
    return res if emit_px else res[0]


def _conv2_pool_sc(hp, px, w9, b, ws, bs, cout, *, bi, ct, split=0,
                   pool_sc=False):
    N, Hp, Wp, c1 = hp.shape
    cin = px.shape[-1]
    cw = w9.shape[1]
    ctw = w9.shape[2] if split else ct
    Ho, Wo = (Hp - 2) // 2 + 2, (Wp - 2) // 2 + 2
    if pool_sc:
        px_spec = pl.BlockSpec((bi, Hp, Wp, cin), lambda i, j: (i, 0, 0, 0))
    else:
        px_spec = pl.BlockSpec((bi, Ho - 2, Wo - 2, cin),
                               lambda i, j: (i, 0, 0, 0))
    grid = (N // bi, cout // ct)
    return pl.pallas_call(
        functools.partial(_conv2_pool_sc_kernel, split=split, pool_sc=pool_sc),
        out_shape=jax.ShapeDtypeStruct((N, Ho, Wo, cout), jnp.bfloat16),
        grid=grid,
        in_specs=[
            pl.BlockSpec((bi, Hp, Wp, c1), lambda i, j: (i, 0, 0, 0)),
            px_spec,
            pl.BlockSpec((9, cw, ctw), lambda i, j: (0, 0, j)),
            pl.BlockSpec((1, ct), lambda i, j: (0, j)),
            pl.BlockSpec((cin, ct), lambda i, j: (0, j)),
            pl.BlockSpec((1, ct), lambda i, j: (0, j)),
        ],
        out_specs=pl.BlockSpec((bi, Ho, Wo, ct), lambda i, j: (i, 0, 0, j)),
        compiler_params=pltpu.CompilerParams(
            dimension_semantics=("parallel", "parallel"), **_VMEM),
    )(hp, px, w9, b, ws, bs)


def _conv2_id_sum(hp, xp, w9, b, cout, *, bi, ct):
    N, Hp, Wp, c1 = hp.shape
    grid = (N // bi, cout // ct)
    return pl.pallas_call(
        _conv2_id_sum_kernel,
        out_shape=jax.ShapeDtypeStruct((N, cout), jnp.float32),
        grid=grid,
        in_specs=[
            pl.BlockSpec((bi, Hp, Wp, c1), lambda i, j: (i, 0, 0, 0)),
            pl.BlockSpec((bi, Hp, Wp, ct), lambda i, j: (i, 0, 0, j)),
            pl.BlockSpec((9, c1, ct), lambda i, j: (0, 0, j)),
            pl.BlockSpec((1, ct), lambda i, j: (0, j)),
        ],
        out_specs=pl.BlockSpec((bi, ct), lambda i, j: (i, j)),
        compiler_params=pltpu.CompilerParams(
            dimension_semantics=("parallel", "parallel"), **_VMEM),
    )(hp, xp, w9, b)


def _w9(wm, cin):
    return wm.reshape(9, cin, wm.shape[-1])


def _row(b):
    return b.reshape(1, -1).astype(jnp.float32)


def _rowp(b):
    r = b.reshape(1, -1).astype(jnp.float32)
    return jnp.concatenate([r, r], axis=1)


def _bd3(w9):
    _, ci, co = w9.shape
    z = jnp.zeros((9, 2 * ci, 2 * co), w9.dtype)
    return z.at[:, :ci, :co].set(w9).at[:, ci:, co:].set(w9)


def _bd2(w):
    ci, co = w.shape
    z = jnp.zeros((2 * ci, 2 * co), w.dtype)
    return z.at[:ci, :co].set(w).at[ci:, co:].set(w)


def kernel(rd1_w1m, rd1_b1, rd1_w2m, rd1_b2, rd1_wsm, rd1_bs,
           rd2_w1m, rd2_b1, rd2_w2m, rd2_b2, rd2_wsm, rd2_bs,
           rd3_w1m, rd3_b1, rd3_w2m, rd3_b2, rd3_wsm, rd3_bs,
           rd4_w1m, rd4_b1, rd4_w2m, rd4_b2, rd4_wsm, rd4_bs,
           rd5_w1m, rd5_b1, rd5_w2m, rd5_b2, rd5_wsm, rd5_bs,
           rd6_w1m, rd6_b1, rd6_w2m, rd6_b2,
           linear_w, linear_b, proj_w, proj_b,
           aux1_w, aux1_b, aux2_w, aux2_b,
           x_src, x_tgt, y):
    B = x_src.shape[0]
    xs = jnp.transpose(x_src, (0, 2, 3, 1)).astype(jnp.bfloat16)
    xt = jnp.transpose(x_tgt, (0, 2, 3, 1)).astype(jnp.bfloat16)
    x = jnp.concatenate([xs, xt], axis=3)
    xp0 = jnp.pad(x, ((0, 0), (1, 1), (1, 1), (0, 0)))

    h, px = _conv1(xp0, _bd3(_w9(rd1_w1m, 3)), _rowp(rd1_b1), 128,
                   bi=4, ct=128, pre_relu=False, emit_px=True)
    o1 = _conv2_pool_sc(h, px, _bd3(_w9(rd1_w2m, 64)),
                        _rowp(rd1_b2), _bd2(rd1_wsm), _rowp(rd1_bs),
                        128, bi=4, ct=128)

    h, px = _conv1(o1, _bd3(_w9(rd2_w1m, 64)), _rowp(rd2_b1), 256,
                   bi=8, ct=256, pre_relu=True, emit_px=True)
    o2 = _conv2_pool_sc(h, px, _bd3(_w9(rd2_w2m, 128)),
                        _rowp(rd2_b2), _bd2(rd2_wsm), _rowp(rd2_bs),
                        256, bi=8, ct=256)

    h, px = _conv1(o2, _bd3(_w9(rd3_w1m, 128)), _rowp(rd3_b1), 512,
                   bi=8, ct=512, pre_relu=True, emit_px=True)
    o3 = _conv2_pool_sc(h, px, _w9(rd3_w2m, 256),
                        _rowp(rd3_b2), _bd2(rd3_wsm), _rowp(rd3_bs),
                        512, bi=8, ct=512, split=256)

    d = o3[..., 256:] - o3[..., :256]

    h = _conv1(d, _w9(rd4_w1m, 256), _row(rd4_b1), 512,
               bi=16, ct=256, pre_relu=True)
    o4 = _conv2_pool_sc(h, d, _w9(rd4_w2m, 512), _row(rd4_b2),
                        rd4_wsm, _row(rd4_bs), 512, bi=16, ct=256,
                        pool_sc=True)

    h = _conv1(o4, _w9(rd5_w1m, 512), _row(rd5_b1), 1024,
               bi=16, ct=256, pre_relu=True)
    o5 = _conv2_pool_sc(h, o4, _w9(rd5_w2m, 1024), _row(rd5_b2),
                        rd5_wsm, _row(rd5_bs), 1024, bi=16, ct=256,
                        pool_sc=True)

    h = _conv1(o5, _w9(rd6_w1m, 1024), _row(rd6_b1), 1024,
               bi=32, ct=256, pre_relu=True)
    xpool = _conv2_id_sum(h, o5, _w9(rd6_w2m, 1024), _row(rd6_b2), 1024,
                          bi=32, ct=256)

    adv, aux = pl.pallas_call(
        _head_kernel,
        out_shape=(jax.ShapeDtypeStruct((B, 1), jnp.float32),
                   jax.ShapeDtypeStruct((B, aux2_w.shape[1]), jnp.float32)),
        compiler_params=pltpu.CompilerParams(**_VMEM),
    )(xpool, y, linear_w, linear_b, proj_w, proj_b,
      aux1_w, aux1_b, aux2_w, aux2_b)
    return adv, aux


# speedup vs baseline: 1.3268x; 1.1810x over previous
import functools

import jax
import jax.numpy as jnp
from jax.experimental import pallas as pl
from jax.experimental.pallas import tpu as pltpu

_VMEM = dict(vmem_limit_bytes=100 * 1024 * 1024)


def _pad_hw(x):
    bn, H, W, C = x.shape
    zc = jnp.zeros((bn, H, 1, C), x.dtype)
    x = jnp.concatenate([zc, x, zc], axis=2)
    zr = jnp.zeros((bn, 1, W + 2, C), x.dtype)
    return jnp.concatenate([zr, x, zr], axis=1)


def _pool2(x):
    bn, H, W, C = x.shape
    x = x.reshape(bn, H // 2, 2, W, C)
    x = x[:, :, 0] + x[:, :, 1]
    x = x.reshape(bn, H // 2, W // 2, 2, C)
    return (x[:, :, :, 0] + x[:, :, :, 1]) * 0.25


def _tap_matmuls_val(x, w_ref, *, split=0):
    bn, Hp, Wp, Cin = x.shape
    H, W = Hp - 2, Wp - 2
    acc = None
    for t in range(9):
        dh, dw = divmod(t, 3)
        a = x[:, dh:dh + H, dw:dw + W, :].reshape(bn * H * W, Cin)
        if split:
            d = jnp.concatenate(
                [jnp.dot(a[:, :split], w_ref[t],
                         preferred_element_type=jnp.float32),
                 jnp.dot(a[:, split:], w_ref[t],
                         preferred_element_type=jnp.float32)], axis=1)
        else:
            d = jnp.dot(a, w_ref[t], preferred_element_type=jnp.float32)
        acc = d if acc is None else acc + d
    return acc


def _block_kernel(xp_ref, w1_ref, b1_ref, w2_ref, b2_ref, ws_ref, bs_ref,
                  o_ref, *, pre_relu, split2):
    bn, Hp, Wp, Cin = xp_ref.shape
    H, W = Hp - 2, Wp - 2
    c1 = w1_ref.shape[2]
    ct = o_ref.shape[-1]
    x = xp_ref[...]
    xr = jnp.maximum(x, 0) if pre_relu else x
    a1 = _tap_matmuls_val(xr, w1_ref) + b1_ref[...]
    h1 = jnp.maximum(a1, 0.0).astype(jnp.bfloat16)
    h1p = _pad_hw(h1.reshape(bn, H, W, c1))
    a2 = _tap_matmuls_val(h1p, w2_ref, split=split2) + b2_ref[...]
    h = _pool2(a2.reshape(bn, H, W, ct))
    px = _pool2(x[:, 1:H + 1, 1:W + 1, :].astype(jnp.float32))
    px = px.astype(jnp.bfloat16).reshape(bn * (H // 2) * (W // 2), Cin)
    sc = jnp.dot(px, ws_ref[...], preferred_element_type=jnp.float32) \
        + bs_ref[...]
    out = (h.reshape(-1, ct) + sc).astype(jnp.bfloat16)
    o_ref[...] = _pad_hw(out.reshape(bn, H // 2, W // 2, ct))


def _res_block(xp, w1, b1, w2, b2, ws, bs, cout, *, bi, pre_relu, split2=0):
    N, Hp, Wp, cin = xp.shape
    Ho, Wo = (Hp - 2) // 2 + 2, (Wp - 2) // 2 + 2
    return pl.pallas_call(
        functools.partial(_block_kernel, pre_relu=pre_relu, split2=split2),
        out_shape=jax.ShapeDtypeStruct((N, Ho, Wo, cout), jnp.bfloat16),
        grid=(N // bi,),
        in_specs=[
            pl.BlockSpec((bi, Hp, Wp, cin), lambda i: (i, 0, 0, 0)),
            pl.BlockSpec(w1.shape, lambda i: (0, 0, 0)),
            pl.BlockSpec((1, w1.shape[2]), lambda i: (0, 0)),
            pl.BlockSpec(w2.shape, lambda i: (0, 0, 0)),
            pl.BlockSpec((1, cout), lambda i: (0, 0)),
            pl.BlockSpec(ws.shape, lambda i: (0, 0)),
            pl.BlockSpec((1, cout), lambda i: (0, 0)),
        ],
        out_specs=pl.BlockSpec((bi, Ho, Wo, cout), lambda i: (i, 0, 0, 0)),
        compiler_params=pltpu.CompilerParams(
            dimension_semantics=("parallel",), **_VMEM),
    )(xp, w1, b1, w2, b2, ws, bs)


def _tap_matmuls(xp_ref, w_ref, *, pre_relu, split=0):
    bn, Hp, Wp, Cin = xp_ref.shape
    H, W = Hp - 2, Wp - 2
    acc = None
    for t in range(9):
        dh, dw = divmod(t, 3)
        a = xp_ref[:, dh:dh + H, dw:dw + W, :]
        if pre_relu:
            a = jnp.maximum(a, 0)
        a = a.reshape(bn * H * W, Cin)
        if split:
            d = jnp.concatenate(
                [jnp.dot(a[:, :split], w_ref[t],
                         preferred_element_type=jnp.float32),
                 jnp.dot(a[:, split:], w_ref[t],
                         preferred_element_type=jnp.float32)], axis=1)
        else:
            d = jnp.dot(a, w_ref[t], preferred_element_type=jnp.float32)
        acc = d if acc is None else acc + d
    return acc


def _stem_conv_kernel(xp_ref, w_ref, b_ref, o_ref, px_ref):
    bn, Hp, Wp, Cin = xp_ref.shape
    H, W = Hp - 2, Wp - 2
    x = xp_ref[...]
    bands = jnp.concatenate([x[:, dh:dh + H] for dh in range(3)], axis=3)
    acc = None
    for dw in range(3):
        a = bands[:, :, dw:dw + W, :].reshape(bn * H * W, 3 * Cin)
        d = jnp.dot(a, w_ref[dw], preferred_element_type=jnp.float32)
        acc = d if acc is None else acc + d
    acc = acc + b_ref[...]
    out = jnp.maximum(acc, 0.0).astype(jnp.bfloat16)
    o_ref[...] = _pad_hw(out.reshape(bn, H, W, -1))
    xin = x[:, 1:H + 1, 1:W + 1, :].astype(jnp.float32)
    px_ref[...] = _pool2(xin).astype(jnp.bfloat16)


def _stem_conv(xp, w18, b, cout, *, bi):
    N, Hp, Wp, cin = xp.shape
    Ho, Wo = (Hp - 2) // 2, (Wp - 2) // 2
    return pl.pallas_call(
        _stem_conv_kernel,
        out_shape=(jax.ShapeDtypeStruct((N, Hp, Wp, cout), jnp.bfloat16),
                   jax.ShapeDtypeStruct((N, Ho, Wo, cin), jnp.bfloat16)),
        grid=(N // bi,),
        in_specs=[
            pl.BlockSpec((bi, Hp, Wp, cin), lambda i: (i, 0, 0, 0)),
            pl.BlockSpec(w18.shape, lambda i: (0, 0, 0)),
            pl.BlockSpec((1, cout), lambda i: (0, 0)),
        ],
        out_specs=(
            pl.BlockSpec((bi, Hp, Wp, cout), lambda i: (i, 0, 0, 0)),
            pl.BlockSpec((bi, Ho, Wo, cin), lambda i: (i, 0, 0, 0)),
        ),
        compiler_params=pltpu.CompilerParams(
            dimension_semantics=("parallel",), **_VMEM),
    )(xp, w18, b)


def _conv1_kernel(xp_ref, w_ref, b_ref, *out_refs, pre_relu):
    o_ref = out_refs[0]
    px_ref = out_refs[1] if len(out_refs) > 1 else None
    bn, Hp, Wp, _ = xp_ref.shape
    H, W = Hp - 2, Wp - 2
    acc = _tap_matmuls(xp_ref, w_ref, pre_relu=pre_relu) + b_ref[...]
    out = jnp.maximum(acc, 0.0).astype(jnp.bfloat16)
    o_ref[...] = _pad_hw(out.reshape(bn, H, W, -1))
    if px_ref is not None:
        xin = xp_ref[:, 1:H + 1, 1:W + 1, :].astype(jnp.float32)
        px_ref[...] = _pool2(xin).astype(jnp.bfloat16)


def _conv2_pool_sc_kernel(hp_ref, px_ref, w_ref, b_ref, ws_ref, bs_ref,
                          o_ref, *, split, pool_sc):
    bn, Hp, Wp, _ = hp_ref.shape
    H, W = Hp - 2, Wp - 2
    ct = o_ref.shape[-1]
    acc = _tap_matmuls(hp_ref, w_ref, pre_relu=False, split=split) + b_ref[...]
    h = _pool2(acc.reshape(bn, H, W, ct))
    if pool_sc:
        xin = px_ref[:, 1:H + 1, 1:W + 1, :].astype(jnp.float32)
        px = _pool2(xin).astype(jnp.bfloat16)
    else:
        px = px_ref[...]
    cin = px.shape[-1]
    sc = jnp.dot(px.reshape(bn * (H // 2) * (W // 2), cin), ws_ref[...],
                 preferred_element_type=jnp.float32) + bs_ref[...]
    out = (h.reshape(-1, ct) + sc).astype(jnp.bfloat16)
    o_ref[...] = _pad_hw(out.reshape(bn, H // 2, W // 2, ct))


def _conv2_id_sum_kernel(hp_ref, xp_ref, w_ref, b_ref, o_ref):
    bn, Hp, Wp, _ = hp_ref.shape
    H, W = Hp - 2, Wp - 2
    ct = o_ref.shape[-1]
    acc = _tap_matmuls(hp_ref, w_ref, pre_relu=False) + b_ref[...]
    xin = xp_ref[:, 1:H + 1, 1:W + 1, :].astype(jnp.float32)
    s = jnp.maximum(acc + xin.reshape(bn * H * W, ct), 0.0)
    o_ref[...] = jnp.sum(s.reshape(bn, H * W, ct), axis=1)


def _head_kernel(x_ref, y_ref, wl_ref, bl_ref, wp_ref, bp_ref,
                 wa1_ref, ba1_ref, wa2_ref, ba2_ref, adv_ref, aux_ref):
    x = x_ref[...]
    adv = jnp.sum(x * wl_ref[...], axis=1, keepdims=True) + bl_ref[...]
    yp = jnp.dot(y_ref[...], wp_ref[...],
                 preferred_element_type=jnp.float32) + bp_ref[...]
    adv = adv + jnp.sum(x * yp, axis=1, keepdims=True)
    adv_ref[...] = adv
    h = jnp.maximum(
        jnp.dot(x, wa1_ref[...], preferred_element_type=jnp.float32)
        + ba1_ref[...], 0.0)
    aux_ref[...] = jnp.dot(h, wa2_ref[...],
                           preferred_element_type=jnp.float32) + ba2_ref[...]


def _conv1(xp, w9, b, cout, *, bi, ct, pre_relu, emit_px=False):
    N, Hp, Wp, cin = xp.shape
    cw = w9.shape[1]
    grid = (N // bi, cout // ct)
    out_shape = [jax.ShapeDtypeStruct((N, Hp, Wp, cout), jnp.bfloat16)]
    out_specs = [pl.BlockSpec((bi, Hp, Wp, ct), lambda i, j: (i, 0, 0, j))]
    if emit_px:
        Ho, Wo = (Hp - 2) // 2, (Wp - 2) // 2
        out_shape.append(jax.ShapeDtypeStruct((N, Ho, Wo, cin), jnp.bfloat16))
        out_specs.append(pl.BlockSpec((bi, Ho, Wo, cin),
                                      lambda i, j: (i, 0, 0, 0)))
    res = pl.pallas_call(
        functools.partial(_conv1_kernel, pre_relu=pre_relu),
        out_shape=tuple(out_shape),
        grid=grid,
        in_specs=[
            pl.BlockSpec((bi, Hp, Wp, cin), lambda i, j: (i, 0, 0, 0)),
            pl.BlockSpec((9, cw, ct), lambda i, j: (0, 0, j)),
            pl.BlockSpec((1, ct), lambda i, j: (0, j)),
        ],
        out_specs=tuple(out_specs),
        compiler_params=pltpu.CompilerParams(
            dimension_semantics=("parallel", "parallel"), **_VMEM),
    )(xp, w9, b)
    return res if emit_px else res[0]


def _conv2_pool_sc(hp, px, w9, b, ws, bs, cout, *, bi, ct, split=0,
                   pool_sc=False):
    N, Hp, Wp, c1 = hp.shape
    cin = px.shape[-1]
    cw = w9.shape[1]
    ctw = w9.shape[2] if split else ct
    Ho, Wo = (Hp - 2) // 2 + 2, (Wp - 2) // 2 + 2
    if pool_sc:
        px_spec = pl.BlockSpec((bi, Hp, Wp, cin), lambda i, j: (i, 0, 0, 0))
    else:
        px_spec = pl.BlockSpec((bi, Ho - 2, Wo - 2, cin),
                               lambda i, j: (i, 0, 0, 0))
    grid = (N // bi, cout // ct)
    return pl.pallas_call(
        functools.partial(_conv2_pool_sc_kernel, split=split, pool_sc=pool_sc),
        out_shape=jax.ShapeDtypeStruct((N, Ho, Wo, cout), jnp.bfloat16),
        grid=grid,
        in_specs=[
            pl.BlockSpec((bi, Hp, Wp, c1), lambda i, j: (i, 0, 0, 0)),
            px_spec,
            pl.BlockSpec((9, cw, ctw), lambda i, j: (0, 0, j)),
            pl.BlockSpec((1, ct), lambda i, j: (0, j)),
            pl.BlockSpec((cin, ct), lambda i, j: (0, j)),
            pl.BlockSpec((1, ct), lambda i, j: (0, j)),
        ],
        out_specs=pl.BlockSpec((bi, Ho, Wo, ct), lambda i, j: (i, 0, 0, j)),
        compiler_params=pltpu.CompilerParams(
            dimension_semantics=("parallel", "parallel"), **_VMEM),
    )(hp, px, w9, b, ws, bs)


def _conv2_id_sum(hp, xp, w9, b, cout, *, bi, ct):
    N, Hp, Wp, c1 = hp.shape
    grid = (N // bi, cout // ct)
    return pl.pallas_call(
        _conv2_id_sum_kernel,
        out_shape=jax.ShapeDtypeStruct((N, cout), jnp.float32),
        grid=grid,
        in_specs=[
            pl.BlockSpec((bi, Hp, Wp, c1), lambda i, j: (i, 0, 0, 0)),
            pl.BlockSpec((bi, Hp, Wp, ct), lambda i, j: (i, 0, 0, j)),
            pl.BlockSpec((9, c1, ct), lambda i, j: (0, 0, j)),
            pl.BlockSpec((1, ct), lambda i, j: (0, j)),
        ],
        out_specs=pl.BlockSpec((bi, ct), lambda i, j: (i, j)),
        compiler_params=pltpu.CompilerParams(
            dimension_semantics=("parallel", "parallel"), **_VMEM),
    )(hp, xp, w9, b)


def _w9(wm, cin):
    return wm.reshape(9, cin, wm.shape[-1])


def _row(b):
    return b.reshape(1, -1).astype(jnp.float32)


def _rowp(b):
    r = b.reshape(1, -1).astype(jnp.float32)
    return jnp.concatenate([r, r], axis=1)


def _bd3(w9):
    _, ci, co = w9.shape
    z = jnp.zeros((9, 2 * ci, 2 * co), w9.dtype)
    return z.at[:, :ci, :co].set(w9).at[:, ci:, co:].set(w9)


def _bd2(w):
    ci, co = w.shape
    z = jnp.zeros((2 * ci, 2 * co), w.dtype)
    return z.at[:ci, :co].set(w).at[ci:, co:].set(w)


def kernel(rd1_w1m, rd1_b1, rd1_w2m, rd1_b2, rd1_wsm, rd1_bs,
           rd2_w1m, rd2_b1, rd2_w2m, rd2_b2, rd2_wsm, rd2_bs,
           rd3_w1m, rd3_b1, rd3_w2m, rd3_b2, rd3_wsm, rd3_bs,
           rd4_w1m, rd4_b1, rd4_w2m, rd4_b2, rd4_wsm, rd4_bs,
           rd5_w1m, rd5_b1, rd5_w2m, rd5_b2, rd5_wsm, rd5_bs,
           rd6_w1m, rd6_b1, rd6_w2m, rd6_b2,
           linear_w, linear_b, proj_w, proj_b,
           aux1_w, aux1_b, aux2_w, aux2_b,
           x_src, x_tgt, y):
    B = x_src.shape[0]
    xs = jnp.transpose(x_src, (0, 2, 3, 1)).astype(jnp.bfloat16)
    xt = jnp.transpose(x_tgt, (0, 2, 3, 1)).astype(jnp.bfloat16)
    x = jnp.concatenate([xs, xt], axis=3)
    xp0 = jnp.pad(x, ((0, 0), (1, 1), (1, 1), (0, 0)))

    w1p = _bd3(_w9(rd1_w1m, 3))
    w18 = jnp.stack([jnp.concatenate([w1p[dh * 3 + dw] for dh in range(3)],
                                     axis=0) for dw in range(3)])
    h, px = _stem_conv(xp0, w18, _rowp(rd1_b1), 128, bi=2)
    o1 = _conv2_pool_sc(h, px, _bd3(_w9(rd1_w2m, 64)),
                        _rowp(rd1_b2), _bd2(rd1_wsm), _rowp(rd1_bs),
                        128, bi=2, ct=128)

    h, px = _conv1(o1, _bd3(_w9(rd2_w1m, 64)), _rowp(rd2_b1), 256,
                   bi=8, ct=256, pre_relu=True, emit_px=True)
    o2 = _conv2_pool_sc(h, px, _bd3(_w9(rd2_w2m, 128)),
                        _rowp(rd2_b2), _bd2(rd2_wsm), _rowp(rd2_bs),
                        256, bi=8, ct=256)

    h, px = _conv1(o2, _bd3(_w9(rd3_w1m, 128)), _rowp(rd3_b1), 512,
                   bi=8, ct=512, pre_relu=True, emit_px=True)
    o3 = _conv2_pool_sc(h, px, _w9(rd3_w2m, 256),
                        _rowp(rd3_b2), _bd2(rd3_wsm), _rowp(rd3_bs),
                        512, bi=8, ct=512, split=256)

    d = o3[..., 256:] - o3[..., :256]

    h = _conv1(d, _w9(rd4_w1m, 256), _row(rd4_b1), 512,
               bi=16, ct=256, pre_relu=True)
    o4 = _conv2_pool_sc(h, d, _w9(rd4_w2m, 512), _row(rd4_b2),
                        rd4_wsm, _row(rd4_bs), 512, bi=16, ct=256,
                        pool_sc=True)

    h = _conv1(o4, _w9(rd5_w1m, 512), _row(rd5_b1), 1024,
               bi=16, ct=256, pre_relu=True)
    o5 = _conv2_pool_sc(h, o4, _w9(rd5_w2m, 1024), _row(rd5_b2),
                        rd5_wsm, _row(rd5_bs), 1024, bi=16, ct=256,
                        pool_sc=True)

    h = _conv1(o5, _w9(rd6_w1m, 1024), _row(rd6_b1), 1024,
               bi=32, ct=256, pre_relu=True)
    xpool = _conv2_id_sum(h, o5, _w9(rd6_w2m, 1024), _row(rd6_b2), 1024,
                          bi=32, ct=256)

    adv, aux = pl.pallas_call(
        _head_kernel,
        out_shape=(jax.ShapeDtypeStruct((B, 1), jnp.float32),
                   jax.ShapeDtypeStruct((B, aux2_w.shape[1]), jnp.float32)),
        compiler_params=pltpu.CompilerParams(**_VMEM),
    )(xpool, y, linear_w, linear_b, proj_w, proj_b,
      aux1_w, aux1_b, aux2_w, aux2_b)
    return adv, aux
